# baseline (speedup 1.0000x reference)
import jax
import jax.numpy as jnp
from jax.experimental import pallas as pl
from jax.experimental.pallas import tpu as pltpu

_BB = 8


def _fwd_kernel(x_ref, adj_ref, act_ref, embW_ref, embb_ref, WextT_ref,
                gbT_ref, W1T_ref, b1T_ref, W2T_ref, b2T_ref, q_ref):
    BB, N, F = x_ref.shape
    R = adj_ref.shape[1]
    NAG, A = act_ref.shape[1], act_ref.shape[2]
    H = embW_ref.shape[1]

    x = x_ref[...].reshape(BB * N, F)
    h = jnp.dot(x, embW_ref[...],
                preferred_element_type=jnp.float32) + embb_ref[...]
    hT = jnp.transpose(h.astype(jnp.bfloat16))
    hrxT = jnp.dot(WextT_ref[...], hT,
                   preferred_element_type=jnp.float32)
    hrxTb = hrxT.astype(jnp.bfloat16)

    og = jax.lax.broadcasted_iota(jnp.int32, (R, R * N), 0)
    oi = jax.lax.broadcasted_iota(jnp.int32, (R, R * N), 1)
    ones_blk = (oi // N == og).astype(jnp.bfloat16)

    xgTs = []
    for b in range(BB):
        cols = slice(b * N, (b + 1) * N)
        Ab = adj_ref[b].astype(jnp.bfloat16).reshape(R * N, N)
        cnts = jnp.dot(ones_blk, Ab,
                       preferred_element_type=jnp.float32)
        rc = 1.0 / jnp.maximum(cnts, 1.0)
        accT = hrxT[R * H:(R + 1) * H, cols] + gbT_ref[...]
        for r in range(R):
            SaT = jnp.dot(hrxTb[r * H:(r + 1) * H, cols],
                          Ab[r * N:(r + 1) * N],
                          preferred_element_type=jnp.float32)
            accT = accT + SaT * rc[r:r + 1]
        outT = jnp.maximum(accT, 0.0)
        xgTs.append(jnp.max(outT, axis=1, keepdims=True))
    xgT = jnp.concatenate(xgTs, axis=1)

    iota_r = jax.lax.broadcasted_iota(jnp.int32, (BB, A), 1)
    iota_c = jax.lax.broadcasted_iota(jnp.int32, (A, BB), 0)
    firsts = []
    for a in range(NAG):
        act = act_ref[:, a, :]
        mx = jnp.max(act, axis=1, keepdims=True)
        firsts.append(jnp.min(jnp.where(act >= mx, iota_r, A),
                              axis=1, keepdims=True))
    firstsT = jnp.transpose(jnp.concatenate(firsts, axis=1))

    qTs = []
    for a in range(NAG):
        h1T = jnp.dot(W1T_ref[a], xgT,
                      preferred_element_type=jnp.float32) + b1T_ref[:, a:a + 1]
        h1T = jnp.where(h1T >= 0, h1T, 0.01 * h1T)
        allqT = jnp.dot(W2T_ref[a], h1T,
                        preferred_element_type=jnp.float32) + b2T_ref[:, a:a + 1]
        sel = iota_c == firstsT[a:a + 1, :]
        qTs.append(jnp.sum(jnp.where(sel, allqT, 0.0),
                           axis=0, keepdims=True))
    q_ref[:, 0, :] = jnp.transpose(jnp.concatenate(qTs, axis=0))


def kernel(unary_tensor, binary_tensor, actions, emb_W, emb_b, W_rel, W_root,
           g_b, c_W1, c_b1, c_W2, c_b2):
    B, N, F = unary_tensor.shape
    R = binary_tensor.shape[3]
    NAG, _, A = actions.shape
    H = emb_W.shape[1]

    adj = jnp.transpose(binary_tensor, (0, 3, 1, 2)).astype(jnp.int8)
    act = jnp.transpose(actions, (1, 0, 2))
    emb_b2 = emb_b.reshape(1, H)

    WextT = jnp.concatenate(
        [jnp.transpose(W_rel, (0, 2, 1)).reshape(R * H, H), W_root.T], axis=0)
    gbT = g_b.reshape(H, 1)

    c_W1T = jnp.transpose(c_W1, (0, 2, 1))
    c_W2T = jnp.transpose(c_W2, (0, 2, 1))
    c_b1T = c_b1.T
    c_b2T = c_b2.T

    q3 = pl.pallas_call(
        _fwd_kernel,
        grid=(B // _BB,),
        in_specs=[
            pl.BlockSpec((_BB, N, F), lambda b: (b, 0, 0)),
            pl.BlockSpec((_BB, R, N, N), lambda b: (b, 0, 0, 0)),
            pl.BlockSpec((_BB, NAG, A), lambda b: (b, 0, 0)),
            pl.BlockSpec((F, H), lambda b: (0, 0)),
            pl.BlockSpec((1, H), lambda b: (0, 0)),
            pl.BlockSpec(((R + 1) * H, H), lambda b: (0, 0)),
            pl.BlockSpec((H, 1), lambda b: (0, 0)),
            pl.BlockSpec((NAG, H, H), lambda b: (0, 0, 0)),
            pl.BlockSpec((H, NAG), lambda b: (0, 0)),
            pl.BlockSpec((NAG, A, H), lambda b: (0, 0, 0)),
            pl.BlockSpec((A, NAG), lambda b: (0, 0)),
        ],
        out_specs=pl.BlockSpec((_BB, 1, NAG), lambda b: (b, 0, 0)),
        out_shape=jax.ShapeDtypeStruct((B, 1, NAG), jnp.float32),
        compiler_params=pltpu.CompilerParams(
            dimension_semantics=("parallel",)),
    )(unary_tensor.astype(jnp.bfloat16), adj, act,
      emb_W.astype(jnp.bfloat16), emb_b2, WextT.astype(jnp.bfloat16), gbT,
      c_W1T, c_b1T, c_W2T, c_b2T)

    return q3.reshape(B, NAG).T[:, :, None]


# speedup vs baseline: 1.0651x; 1.0651x over previous
import jax
import jax.numpy as jnp
from jax.experimental import pallas as pl
from jax.experimental.pallas import tpu as pltpu

_BB = 8


def _fwd_kernel(x_ref, adj_ref, act_ref, embW_ref, embb_ref, WextT_ref,
                gbT_ref, W1_ref, b1_ref, W2_ref, b2_ref, q_ref):
    BB, N, F = x_ref.shape
    R = adj_ref.shape[1]
    NAG, _, A = act_ref.shape
    H = embW_ref.shape[1]

    x = x_ref[...].reshape(BB * N, F).astype(jnp.bfloat16)
    h = jnp.dot(x, embW_ref[...],
                preferred_element_type=jnp.float32) + embb_ref[...]
    hT = jnp.transpose(h.astype(jnp.bfloat16))
    hrxT = jnp.dot(WextT_ref[...], hT,
                   preferred_element_type=jnp.float32)
    hrxTb = hrxT.astype(jnp.bfloat16)

    og = jax.lax.broadcasted_iota(jnp.int32, (R, R * N), 0)
    oi = jax.lax.broadcasted_iota(jnp.int32, (R, R * N), 1)
    ones_blk = (oi // N == og).astype(jnp.bfloat16)

    xgTs = []
    for b in range(BB):
        cols = slice(b * N, (b + 1) * N)
        Ab = adj_ref[b].astype(jnp.bfloat16).reshape(R * N, N)
        cnts = jnp.dot(ones_blk, Ab,
                       preferred_element_type=jnp.float32)
        rc = 1.0 / jnp.maximum(cnts, 1.0)
        accT = hrxT[R * H:(R + 1) * H, cols] + gbT_ref[...]
        for r in range(R):
            SaT = jnp.dot(hrxTb[r * H:(r + 1) * H, cols],
                          Ab[r * N:(r + 1) * N],
                          preferred_element_type=jnp.float32)
            accT = accT + SaT * rc[r:r + 1]
        outT = jnp.maximum(accT, 0.0)
        xgTs.append(jnp.max(outT, axis=1, keepdims=True))
    xg = jnp.transpose(jnp.concatenate(xgTs, axis=1))

    iota = jax.lax.broadcasted_iota(jnp.int32, (BB, A), 1)
    for a in range(NAG):
        h1 = jnp.dot(xg, W1_ref[a],
                     preferred_element_type=jnp.float32) + b1_ref[a:a + 1]
        h1 = jnp.where(h1 >= 0, h1, 0.01 * h1)
        allq = jnp.dot(h1, W2_ref[a],
                       preferred_element_type=jnp.float32) + b2_ref[a:a + 1]
        act = act_ref[a]
        mx = jnp.max(act, axis=1, keepdims=True)
        first = jnp.min(jnp.where(act >= mx, iota, A),
                        axis=1, keepdims=True)
        q = jnp.sum(jnp.where(iota == first, allq, 0.0),
                    axis=1, keepdims=True)
        q_ref[:, 0, a:a + 1] = q


def kernel(unary_tensor, binary_tensor, actions, emb_W, emb_b, W_rel, W_root,
           g_b, c_W1, c_b1, c_W2, c_b2):
    B, N, F = unary_tensor.shape
    R = binary_tensor.shape[3]
    NAG, _, A = actions.shape
    H = emb_W.shape[1]

    adj = jnp.transpose(binary_tensor, (0, 3, 1, 2)).astype(jnp.int8)
    emb_b2 = emb_b.reshape(1, H)
    WextT = jnp.concatenate(
        [jnp.transpose(W_rel, (0, 2, 1)).reshape(R * H, H), W_root.T],
        axis=0).astype(jnp.bfloat16)
    gbT = g_b.reshape(H, 1)

    q3 = pl.pallas_call(
        _fwd_kernel,
        grid=(B // _BB,),
        in_specs=[
            pl.BlockSpec((_BB, N, F), lambda b: (b, 0, 0)),
            pl.BlockSpec((_BB, R, N, N), lambda b: (b, 0, 0, 0)),
            pl.BlockSpec((NAG, _BB, A), lambda b: (0, b, 0)),
            pl.BlockSpec((F, H), lambda b: (0, 0)),
            pl.BlockSpec((1, H), lambda b: (0, 0)),
            pl.BlockSpec(((R + 1) * H, H), lambda b: (0, 0)),
            pl.BlockSpec((H, 1), lambda b: (0, 0)),
            pl.BlockSpec((NAG, H, H), lambda b: (0, 0, 0)),
            pl.BlockSpec((NAG, H), lambda b: (0, 0)),
            pl.BlockSpec((NAG, H, A), lambda b: (0, 0, 0)),
            pl.BlockSpec((NAG, A), lambda b: (0, 0)),
        ],
        out_specs=pl.BlockSpec((_BB, 1, NAG), lambda b: (b, 0, 0)),
        out_shape=jax.ShapeDtypeStruct((B, 1, NAG), jnp.float32),
        compiler_params=pltpu.CompilerParams(
            dimension_semantics=("parallel",)),
    )(unary_tensor, adj, actions,
      emb_W.astype(jnp.bfloat16), emb_b2, WextT, gbT,
      c_W1, c_b1, c_W2, c_b2)

    return q3.reshape(B, NAG).T[:, :, None]


# speedup vs baseline: 1.2778x; 1.1998x over previous
import jax
import jax.numpy as jnp
from jax.experimental import pallas as pl
from jax.experimental.pallas import tpu as pltpu

_BB = 16


def _fwd_kernel(x_ref, adj_ref, act_ref, embW_ref, embb_ref, WextT_ref,
                gbT_ref, W1_ref, b1_ref, W2_ref, b2_ref, q_ref):
    BB, N, F = x_ref.shape
    R = adj_ref.shape[1]
    NAG, _, A = act_ref.shape
    H = embW_ref.shape[1]

    x = x_ref[...].reshape(BB * N, F).astype(jnp.bfloat16)
    h = jnp.dot(x, embW_ref[...],
                preferred_element_type=jnp.float32) + embb_ref[...]
    hT = jnp.transpose(h.astype(jnp.bfloat16))
    hrxT = jnp.dot(WextT_ref[...], hT,
                   preferred_element_type=jnp.float32)
    hrxTb = hrxT.astype(jnp.bfloat16)

    og = jax.lax.broadcasted_iota(jnp.int32, (R, R * N), 0)
    oi = jax.lax.broadcasted_iota(jnp.int32, (R, R * N), 1)
    ones_blk = (oi // N == og).astype(jnp.bfloat16)

    xgTs = []
    for b in range(BB):
        cols = slice(b * N, (b + 1) * N)
        Ab = adj_ref[b].astype(jnp.bfloat16).reshape(R * N, N)
        cnts = jnp.dot(ones_blk, Ab,
                       preferred_element_type=jnp.float32)
        rc = 1.0 / jnp.maximum(cnts, 1.0)
        accT = hrxT[R * H:(R + 1) * H, cols] + gbT_ref[...]
        for r in range(R):
            SaT = jnp.dot(hrxTb[r * H:(r + 1) * H, cols],
                          Ab[r * N:(r + 1) * N],
                          preferred_element_type=jnp.float32)
            accT = accT + SaT * rc[r:r + 1]
        outT = jnp.maximum(accT, 0.0)
        xgTs.append(jnp.max(outT, axis=1, keepdims=True))
    xg = jnp.transpose(jnp.concatenate(xgTs, axis=1))

    iota = jax.lax.broadcasted_iota(jnp.int32, (BB, A), 1)
    for a in range(NAG):
        h1 = jnp.dot(xg, W1_ref[a],
                     preferred_element_type=jnp.float32) + b1_ref[a:a + 1]
        h1 = jnp.where(h1 >= 0, h1, 0.01 * h1)
        allq = jnp.dot(h1, W2_ref[a],
                       preferred_element_type=jnp.float32) + b2_ref[a:a + 1]
        act = act_ref[a]
        mx = jnp.max(act, axis=1, keepdims=True)
        first = jnp.min(jnp.where(act >= mx, iota, A),
                        axis=1, keepdims=True)
        q = jnp.sum(jnp.where(iota == first, allq, 0.0),
                    axis=1, keepdims=True)
        q_ref[:, 0, a:a + 1] = q


def kernel(unary_tensor, binary_tensor, actions, emb_W, emb_b, W_rel, W_root,
           g_b, c_W1, c_b1, c_W2, c_b2):
    B, N, F = unary_tensor.shape
    R = binary_tensor.shape[3]
    NAG, _, A = actions.shape
    H = emb_W.shape[1]

    adj = jnp.transpose(binary_tensor, (0, 3, 1, 2)).astype(jnp.int8)
    emb_b2 = emb_b.reshape(1, H)
    WextT = jnp.concatenate(
        [jnp.transpose(W_rel, (0, 2, 1)).reshape(R * H, H), W_root.T],
        axis=0).astype(jnp.bfloat16)
    gbT = g_b.reshape(H, 1)

    q3 = pl.pallas_call(
        _fwd_kernel,
        grid=(B // _BB,),
        in_specs=[
            pl.BlockSpec((_BB, N, F), lambda b: (b, 0, 0)),
            pl.BlockSpec((_BB, R, N, N), lambda b: (b, 0, 0, 0)),
            pl.BlockSpec((NAG, _BB, A), lambda b: (0, b, 0)),
            pl.BlockSpec((F, H), lambda b: (0, 0)),
            pl.BlockSpec((1, H), lambda b: (0, 0)),
            pl.BlockSpec(((R + 1) * H, H), lambda b: (0, 0)),
            pl.BlockSpec((H, 1), lambda b: (0, 0)),
            pl.BlockSpec((NAG, H, H), lambda b: (0, 0, 0)),
            pl.BlockSpec((NAG, H), lambda b: (0, 0)),
            pl.BlockSpec((NAG, H, A), lambda b: (0, 0, 0)),
            pl.BlockSpec((NAG, A), lambda b: (0, 0)),
        ],
        out_specs=pl.BlockSpec((_BB, 1, NAG), lambda b: (b, 0, 0)),
        out_shape=jax.ShapeDtypeStruct((B, 1, NAG), jnp.float32),
        compiler_params=pltpu.CompilerParams(
            dimension_semantics=("parallel",)),
    )(unary_tensor, adj, actions,
      emb_W.astype(jnp.bfloat16), emb_b2, WextT, gbT,
      c_W1, c_b1, c_W2, c_b2)

    return q3.reshape(B, NAG).T[:, :, None]


# speedup vs baseline: 1.3778x; 1.0782x over previous
import jax
import jax.numpy as jnp
from jax.experimental import pallas as pl
from jax.experimental.pallas import tpu as pltpu

_BB = 32


def _fwd_kernel(x_ref, adj_ref, act_ref, embW_ref, embb_ref, WextT_ref,
                gbT_ref, W1_ref, b1_ref, W2_ref, b2_ref, q_ref):
    BB, N, F = x_ref.shape
    R = adj_ref.shape[1]
    NAG, _, A = act_ref.shape
    H = embW_ref.shape[1]

    x = x_ref[...].reshape(BB * N, F).astype(jnp.bfloat16)
    h = jnp.dot(x, embW_ref[...],
                preferred_element_type=jnp.float32) + embb_ref[...]
    hT = jnp.transpose(h.astype(jnp.bfloat16))
    hrxT = jnp.dot(WextT_ref[...], hT,
                   preferred_element_type=jnp.float32)
    hrxTb = hrxT.astype(jnp.bfloat16)

    og = jax.lax.broadcasted_iota(jnp.int32, (R, R * N), 0)
    oi = jax.lax.broadcasted_iota(jnp.int32, (R, R * N), 1)
    ones_blk = (oi // N == og).astype(jnp.bfloat16)

    xgTs = []
    for b in range(BB):
        cols = slice(b * N, (b + 1) * N)
        Ab = adj_ref[b].astype(jnp.bfloat16).reshape(R * N, N)
        cnts = jnp.dot(ones_blk, Ab,
                       preferred_element_type=jnp.float32)
        rc = 1.0 / jnp.maximum(cnts, 1.0)
        accT = hrxT[R * H:(R + 1) * H, cols] + gbT_ref[...]
        for r in range(R):
            SaT = jnp.dot(hrxTb[r * H:(r + 1) * H, cols],
                          Ab[r * N:(r + 1) * N],
                          preferred_element_type=jnp.float32)
            accT = accT + SaT * rc[r:r + 1]
        outT = jnp.maximum(accT, 0.0)
        xgTs.append(jnp.max(outT, axis=1, keepdims=True))
    xg = jnp.transpose(jnp.concatenate(xgTs, axis=1))

    iota = jax.lax.broadcasted_iota(jnp.int32, (BB, A), 1)
    for a in range(NAG):
        h1 = jnp.dot(xg, W1_ref[a],
                     preferred_element_type=jnp.float32) + b1_ref[a:a + 1]
        h1 = jnp.where(h1 >= 0, h1, 0.01 * h1)
        allq = jnp.dot(h1, W2_ref[a],
                       preferred_element_type=jnp.float32) + b2_ref[a:a + 1]
        act = act_ref[a]
        mx = jnp.max(act, axis=1, keepdims=True)
        first = jnp.min(jnp.where(act >= mx, iota, A),
                        axis=1, keepdims=True)
        q = jnp.sum(jnp.where(iota == first, allq, 0.0),
                    axis=1, keepdims=True)
        q_ref[:, 0, a:a + 1] = q


def kernel(unary_tensor, binary_tensor, actions, emb_W, emb_b, W_rel, W_root,
           g_b, c_W1, c_b1, c_W2, c_b2):
    B, N, F = unary_tensor.shape
    R = binary_tensor.shape[3]
    NAG, _, A = actions.shape
    H = emb_W.shape[1]

    adj = jnp.transpose(binary_tensor, (0, 3, 1, 2)).astype(jnp.int8)
    emb_b2 = emb_b.reshape(1, H)
    WextT = jnp.concatenate(
        [jnp.transpose(W_rel, (0, 2, 1)).reshape(R * H, H), W_root.T],
        axis=0).astype(jnp.bfloat16)
    gbT = g_b.reshape(H, 1)

    q3 = pl.pallas_call(
        _fwd_kernel,
        grid=(B // _BB,),
        in_specs=[
            pl.BlockSpec((_BB, N, F), lambda b: (b, 0, 0)),
            pl.BlockSpec((_BB, R, N, N), lambda b: (b, 0, 0, 0)),
            pl.BlockSpec((NAG, _BB, A), lambda b: (0, b, 0)),
            pl.BlockSpec((F, H), lambda b: (0, 0)),
            pl.BlockSpec((1, H), lambda b: (0, 0)),
            pl.BlockSpec(((R + 1) * H, H), lambda b: (0, 0)),
            pl.BlockSpec((H, 1), lambda b: (0, 0)),
            pl.BlockSpec((NAG, H, H), lambda b: (0, 0, 0)),
            pl.BlockSpec((NAG, H), lambda b: (0, 0)),
            pl.BlockSpec((NAG, H, A), lambda b: (0, 0, 0)),
            pl.BlockSpec((NAG, A), lambda b: (0, 0)),
        ],
        out_specs=pl.BlockSpec((_BB, 1, NAG), lambda b: (b, 0, 0)),
        out_shape=jax.ShapeDtypeStruct((B, 1, NAG), jnp.float32),
        compiler_params=pltpu.CompilerParams(
            dimension_semantics=("parallel",)),
    )(unary_tensor, adj, actions,
      emb_W.astype(jnp.bfloat16), emb_b2, WextT, gbT,
      c_W1, c_b1, c_W2, c_b2)

    return q3.reshape(B, NAG).T[:, :, None]


# speedup vs baseline: 1.4022x; 1.0178x over previous
import jax
import jax.numpy as jnp
from jax.experimental import pallas as pl
from jax.experimental.pallas import tpu as pltpu

_BB = 64


def _fwd_kernel(x_ref, adj_ref, act_ref, embW_ref, embb_ref, WextT_ref,
                gbT_ref, W1_ref, b1_ref, W2_ref, b2_ref, q_ref):
    BB, N, F = x_ref.shape
    R = adj_ref.shape[1]
    NAG, _, A = act_ref.shape
    H = embW_ref.shape[1]

    x = x_ref[...].reshape(BB * N, F).astype(jnp.bfloat16)
    h = jnp.dot(x, embW_ref[...],
                preferred_element_type=jnp.float32) + embb_ref[...]
    hT = jnp.transpose(h.astype(jnp.bfloat16))
    hrxT = jnp.dot(WextT_ref[...], hT,
                   preferred_element_type=jnp.float32)
    hrxTb = hrxT.astype(jnp.bfloat16)

    og = jax.lax.broadcasted_iota(jnp.int32, (R, R * N), 0)
    oi = jax.lax.broadcasted_iota(jnp.int32, (R, R * N), 1)
    ones_blk = (oi // N == og).astype(jnp.bfloat16)

    xgTs = []
    for b in range(BB):
        cols = slice(b * N, (b + 1) * N)
        Ab = adj_ref[b].astype(jnp.bfloat16).reshape(R * N, N)
        cnts = jnp.dot(ones_blk, Ab,
                       preferred_element_type=jnp.float32)
        rc = 1.0 / jnp.maximum(cnts, 1.0)
        accT = hrxT[R * H:(R + 1) * H, cols] + gbT_ref[...]
        for r in range(R):
            SaT = jnp.dot(hrxTb[r * H:(r + 1) * H, cols],
                          Ab[r * N:(r + 1) * N],
                          preferred_element_type=jnp.float32)
            accT = accT + SaT * rc[r:r + 1]
        outT = jnp.maximum(accT, 0.0)
        xgTs.append(jnp.max(outT, axis=1, keepdims=True))
    xg = jnp.transpose(jnp.concatenate(xgTs, axis=1))

    iota = jax.lax.broadcasted_iota(jnp.int32, (BB, A), 1)
    for a in range(NAG):
        h1 = jnp.dot(xg, W1_ref[a],
                     preferred_element_type=jnp.float32) + b1_ref[a:a + 1]
        h1 = jnp.where(h1 >= 0, h1, 0.01 * h1)
        allq = jnp.dot(h1, W2_ref[a],
                       preferred_element_type=jnp.float32) + b2_ref[a:a + 1]
        act = act_ref[a]
        mx = jnp.max(act, axis=1, keepdims=True)
        first = jnp.min(jnp.where(act >= mx, iota, A),
                        axis=1, keepdims=True)
        q = jnp.sum(jnp.where(iota == first, allq, 0.0),
                    axis=1, keepdims=True)
        q_ref[:, 0, a:a + 1] = q


def kernel(unary_tensor, binary_tensor, actions, emb_W, emb_b, W_rel, W_root,
           g_b, c_W1, c_b1, c_W2, c_b2):
    B, N, F = unary_tensor.shape
    R = binary_tensor.shape[3]
    NAG, _, A = actions.shape
    H = emb_W.shape[1]

    adj = jnp.transpose(binary_tensor, (0, 3, 1, 2)).astype(jnp.int8)
    emb_b2 = emb_b.reshape(1, H)
    WextT = jnp.concatenate(
        [jnp.transpose(W_rel, (0, 2, 1)).reshape(R * H, H), W_root.T],
        axis=0).astype(jnp.bfloat16)
    gbT = g_b.reshape(H, 1)

    q3 = pl.pallas_call(
        _fwd_kernel,
        grid=(B // _BB,),
        in_specs=[
            pl.BlockSpec((_BB, N, F), lambda b: (b, 0, 0)),
            pl.BlockSpec((_BB, R, N, N), lambda b: (b, 0, 0, 0)),
            pl.BlockSpec((NAG, _BB, A), lambda b: (0, b, 0)),
            pl.BlockSpec((F, H), lambda b: (0, 0)),
            pl.BlockSpec((1, H), lambda b: (0, 0)),
            pl.BlockSpec(((R + 1) * H, H), lambda b: (0, 0)),
            pl.BlockSpec((H, 1), lambda b: (0, 0)),
            pl.BlockSpec((NAG, H, H), lambda b: (0, 0, 0)),
            pl.BlockSpec((NAG, H), lambda b: (0, 0)),
            pl.BlockSpec((NAG, H, A), lambda b: (0, 0, 0)),
            pl.BlockSpec((NAG, A), lambda b: (0, 0)),
        ],
        out_specs=pl.BlockSpec((_BB, 1, NAG), lambda b: (b, 0, 0)),
        out_shape=jax.ShapeDtypeStruct((B, 1, NAG), jnp.float32),
        compiler_params=pltpu.CompilerParams(
            dimension_semantics=("parallel",)),
    )(unary_tensor, adj, actions,
      emb_W.astype(jnp.bfloat16), emb_b2, WextT, gbT,
      c_W1, c_b1, c_W2, c_b2)

    return q3.reshape(B, NAG).T[:, :, None]
